# baseline (speedup 1.0000x reference)
import functools

import jax
import jax.numpy as jnp
from jax import lax
from jax.experimental import pallas as pl
from jax.experimental.pallas import tpu as pltpu
from jax.experimental.pallas import tpu_sc as plsc

N_NODES = 10000
D = 128
N_EDGES = 320000

NUM_TILES = 32
CHUNK = 128
CHUNKS_PER_TILE = 80
GROUP = 40
NBUF = 4
E_PAD = NUM_TILES * CHUNKS_PER_TILE * CHUNK
N_ACC = 10240
SLAB = N_ACC // 16
N_AGG = 10112
SLAB_A = N_AGG // 16

_MESH = plsc.VectorSubcoreMesh(core_axis_name="c", subcore_axis_name="s")


def _sc_aggregate(y_bf, srcs, dsts, zrows_bf, ones, with_cnt):
    scratch = [
        pltpu.VMEM((GROUP, CHUNK), jnp.int32),
        pltpu.VMEM((GROUP, CHUNK), jnp.int32),
        pltpu.VMEM((NBUF * CHUNK, D), jnp.bfloat16),
        pltpu.VMEM_SHARED((N_AGG, D), jnp.bfloat16),
        pltpu.VMEM_SHARED((N_AGG, D), jnp.bfloat16),
    ] + [pltpu.SemaphoreType.DMA] * NBUF

    out_type = [jax.ShapeDtypeStruct((2, N_AGG, D), jnp.bfloat16)]
    if with_cnt:
        out_type.append(jax.ShapeDtypeStruct((2, N_AGG, D), jnp.bfloat16))

    @functools.partial(
        pl.kernel, mesh=_MESH, out_type=out_type,
        compiler_params=pltpu.CompilerParams(use_tc_tiling_on_sc=False),
        scratch_types=scratch)
    def run(y_hbm, srcs_hbm, dsts_hbm, zrows_hbm, ones_hbm, *rest):
        if with_cnt:
            agg_hbm, cnt_hbm = rest[0], rest[1]
            src_v, dst_v, rows, y_sh, acc_sh = rest[2:7]
            sems = rest[7:]
        else:
            agg_hbm = rest[0]
            src_v, dst_v, rows, y_sh, acc_sh = rest[1:6]
            sems = rest[6:]
        cid = lax.axis_index("c")
        sid = lax.axis_index("s")
        wid = cid * 16 + sid
        slab = pl.ds(sid * SLAB_A, SLAB_A)

        def sweep(chunk_op):
            def group(g, carry):
                pltpu.sync_copy(srcs_hbm.at[wid, pl.ds(g * GROUP, GROUP)],
                                src_v)
                pltpu.sync_copy(dsts_hbm.at[wid, pl.ds(g * GROUP, GROUP)],
                                dst_v)
                for b in range(NBUF):
                    chunk_op(b, b, False, True)

                def quad(q, c):
                    for b in range(NBUF):
                        chunk_op(NBUF * q + b, b, True, True)
                    return c

                lax.fori_loop(0, GROUP // NBUF - 1, quad, carry)
                for b in range(NBUF):
                    chunk_op(GROUP - NBUF + b, b, True, False)
                return carry

            lax.fori_loop(0, CHUNKS_PER_TILE // GROUP, group, 0)

        if with_cnt:
            pltpu.sync_copy(zrows_hbm, y_sh.at[slab])
            pltpu.sync_copy(zrows_hbm, acc_sh.at[slab])
            pltpu.sync_copy(ones_hbm, rows.at[pl.ds(0, CHUNK)])
            plsc.subcore_barrier()

            def cnt_op(j, b, wait, fire):
                if wait:
                    pltpu.make_async_copy(rows.at[pl.ds(0, CHUNK)],
                                          y_sh.at[dst_v.at[0]],
                                          sems[b]).wait()
                if fire:
                    jn = j if not wait else j + NBUF
                    pltpu.async_copy(rows.at[pl.ds(0, CHUNK)],
                                     y_sh.at[dst_v.at[jn]], sems[b], add=True)

            sweep(cnt_op)
            plsc.subcore_barrier()
            pltpu.sync_copy(y_sh.at[slab], cnt_hbm.at[cid, slab])

        pltpu.sync_copy(y_hbm.at[slab], y_sh.at[slab])
        if not with_cnt:
            pltpu.sync_copy(zrows_hbm, acc_sh.at[slab])
        plsc.subcore_barrier()

        def agg_op(j, b, wait, fire):
            if wait:
                pltpu.make_async_copy(y_sh.at[src_v.at[0]],
                                      rows.at[pl.ds(b * CHUNK, CHUNK)],
                                      sems[b]).wait()
                pltpu.sync_copy(rows.at[pl.ds(b * CHUNK, CHUNK)],
                                acc_sh.at[dst_v.at[j]], add=True)
            if fire:
                jn = j if not wait else j + NBUF
                pltpu.async_copy(y_sh.at[src_v.at[jn]],
                                 rows.at[pl.ds(b * CHUNK, CHUNK)], sems[b])

        sweep(agg_op)
        plsc.subcore_barrier()

        pltpu.sync_copy(acc_sh.at[slab], agg_hbm.at[cid, slab])

    return run(y_bf, srcs, dsts, zrows_bf, ones)


def _root_body(x_ref, wr_ref, b_ref, o_ref):
    o_ref[...] = b_ref[...] + lax.dot_general(
        x_ref[...], wr_ref[...], (((1,), (1,)), ((), ())),
        preferred_element_type=jnp.float32)


def _tc_root(x, Wr, b):
    R = 1000
    return pl.pallas_call(
        _root_body,
        grid=(N_NODES // R,),
        in_specs=[
            pl.BlockSpec((R, D), lambda i: (i, 0)),
            pl.BlockSpec((D, D), lambda i: (0, 0)),
            pl.BlockSpec((1, D), lambda i: (0, 0)),
        ],
        out_specs=pl.BlockSpec((R, D), lambda i: (i, 0)),
        out_shape=jax.ShapeDtypeStruct((N_NODES, D), jnp.float32),
    )(x, Wr, b.reshape(1, D))


def _out_body(relu, bf_out, agg_ref, cnt_ref, xr_ref, wl_ref, *o_refs):
    a = (agg_ref[0].astype(jnp.float32)
         + agg_ref[1].astype(jnp.float32))
    c = (cnt_ref[0, :, 0].astype(jnp.float32)
         + cnt_ref[1, :, 0].astype(jnp.float32))
    mean = a / jnp.maximum(c, 1.0)[:, None]
    h = xr_ref[...] + lax.dot_general(
        mean, wl_ref[...], (((1,), (1,)), ((), ())),
        preferred_element_type=jnp.float32)
    if relu:
        h = jnp.maximum(h, 0.0)
    o_refs[0][...] = h
    if bf_out:
        o_refs[1][...] = h.astype(jnp.bfloat16)


def _tc_dense(agg, cnt, xr, Wl, relu, bf_out):
    R = 1000
    grid = (N_NODES // R,)
    out_shape = [jax.ShapeDtypeStruct((N_NODES, D), jnp.float32)]
    out_specs = [pl.BlockSpec((R, D), lambda i: (i, 0))]
    if bf_out:
        out_shape.append(jax.ShapeDtypeStruct((N_NODES, D), jnp.bfloat16))
        out_specs.append(pl.BlockSpec((R, D), lambda i: (i, 0)))
    return pl.pallas_call(
        functools.partial(_out_body, relu, bf_out),
        grid=grid,
        in_specs=[
            pl.BlockSpec((2, R, D), lambda i: (0, i, 0)),
            pl.BlockSpec((2, R, D), lambda i: (0, i, 0)),
            pl.BlockSpec((R, D), lambda i: (i, 0)),
            pl.BlockSpec((D, D), lambda i: (0, 0)),
        ],
        out_specs=out_specs,
        out_shape=out_shape,
    )(agg, cnt, xr, Wl)


def kernel(x, edge_index, W1l, b1, W1r, W2l, b2, W2r):
    src = edge_index[0].astype(jnp.int32)
    dst = edge_index[1].astype(jnp.int32)
    pad = E_PAD - N_EDGES
    srcs = jnp.concatenate([src, jnp.zeros((pad,), jnp.int32)]).reshape(
        NUM_TILES, CHUNKS_PER_TILE, CHUNK)
    dsts = jnp.concatenate(
        [dst, jnp.full((pad,), N_NODES, jnp.int32)]).reshape(
        NUM_TILES, CHUNKS_PER_TILE, CHUNK)
    ones = jnp.ones((CHUNK, D), jnp.bfloat16)
    zrows_bf = jnp.zeros((SLAB_A, D), jnp.bfloat16)
    rpad = ((0, N_AGG - N_NODES), (0, 0))

    xr1 = _tc_root(x, W1r, b1)
    aggx, cnt = _sc_aggregate(jnp.pad(x.astype(jnp.bfloat16), rpad),
                              srcs, dsts, zrows_bf, ones, True)
    h1, h1bf = _tc_dense(aggx, cnt, xr1, W1l, relu=True, bf_out=True)
    xr2 = _tc_root(h1, W2r, b2)
    (aggh,) = _sc_aggregate(jnp.pad(h1bf, rpad), srcs, dsts, zrows_bf,
                            ones, False)
    (out,) = _tc_dense(aggh, cnt, xr2, W2l, relu=False, bf_out=False)
    return out


# speedup vs baseline: 1.0497x; 1.0497x over previous
import functools

import jax
import jax.numpy as jnp
from jax import lax
from jax.experimental import pallas as pl
from jax.experimental.pallas import tpu as pltpu
from jax.experimental.pallas import tpu_sc as plsc

N_NODES = 10000
D = 128
N_EDGES = 320000

NUM_TILES = 32
CHUNK = 128
CHUNKS_PER_TILE = 80
GROUP = 40
NBUF = 4
E_PAD = NUM_TILES * CHUNKS_PER_TILE * CHUNK
N_AGG = 10112
SLAB_A = N_AGG // 16

_MESH = plsc.VectorSubcoreMesh(core_axis_name="c", subcore_axis_name="s")


def _sc_aggregate(y_bf, srcs, dsts, zrows_bf):
    scratch = [
        pltpu.VMEM((GROUP, CHUNK), jnp.int32),
        pltpu.VMEM((GROUP, CHUNK), jnp.int32),
        pltpu.VMEM((NBUF * CHUNK, D), jnp.bfloat16),
        pltpu.VMEM_SHARED((N_AGG, D), jnp.bfloat16),
        pltpu.VMEM_SHARED((N_AGG, D), jnp.bfloat16),
    ] + [pltpu.SemaphoreType.DMA] * NBUF

    @functools.partial(
        pl.kernel, mesh=_MESH,
        out_type=[jax.ShapeDtypeStruct((2, N_AGG, D), jnp.bfloat16)],
        compiler_params=pltpu.CompilerParams(use_tc_tiling_on_sc=False),
        scratch_types=scratch)
    def run(y_hbm, srcs_hbm, dsts_hbm, zrows_hbm, agg_hbm,
            src_v, dst_v, rows, y_sh, acc_sh, *sems):
        cid = lax.axis_index("c")
        sid = lax.axis_index("s")
        wid = cid * 16 + sid

        pltpu.sync_copy(y_hbm.at[pl.ds(sid * SLAB_A, SLAB_A)],
                        y_sh.at[pl.ds(sid * SLAB_A, SLAB_A)])
        pltpu.sync_copy(zrows_hbm, acc_sh.at[pl.ds(sid * SLAB_A, SLAB_A)])
        plsc.subcore_barrier()

        def fire(j, b):
            pltpu.async_copy(y_sh.at[src_v.at[j]],
                             rows.at[pl.ds(b * CHUNK, CHUNK)], sems[b])

        def half(j, b, prefetch):
            pltpu.make_async_copy(y_sh.at[src_v.at[0]],
                                  rows.at[pl.ds(b * CHUNK, CHUNK)],
                                  sems[b]).wait()
            pltpu.sync_copy(rows.at[pl.ds(b * CHUNK, CHUNK)],
                            acc_sh.at[dst_v.at[j]], add=True)
            if prefetch:
                fire(j + NBUF, b)

        def group(g, carry):
            pltpu.sync_copy(srcs_hbm.at[wid, pl.ds(g * GROUP, GROUP)], src_v)
            pltpu.sync_copy(dsts_hbm.at[wid, pl.ds(g * GROUP, GROUP)], dst_v)
            for b in range(NBUF):
                fire(b, b)

            def quad(q, c):
                for b in range(NBUF):
                    half(NBUF * q + b, b, True)
                return c

            lax.fori_loop(0, GROUP // NBUF - 1, quad, carry)
            for b in range(NBUF):
                half(GROUP - NBUF + b, b, False)
            return carry

        lax.fori_loop(0, CHUNKS_PER_TILE // GROUP, group, 0)
        plsc.subcore_barrier()

        pltpu.sync_copy(acc_sh.at[pl.ds(sid * SLAB_A, SLAB_A)],
                        agg_hbm.at[cid, pl.ds(sid * SLAB_A, SLAB_A)])

    return run(y_bf, srcs, dsts, zrows_bf)[0]


def _cnt_scatter(acc_sh, dst_v, ones_v, j, sem):
    pltpu.async_copy(ones_v, acc_sh.at[dst_v.at[j]], sem, add=True)


def _cnt_wait(acc_sh, dst_v, ones_v, sem):
    pltpu.make_async_copy(ones_v, acc_sh.at[dst_v.at[0]], sem).wait()


def _sc_count(dsts, zrows_bf, ones):
    scratch = [
        pltpu.VMEM((GROUP, CHUNK), jnp.int32),
        pltpu.VMEM((CHUNK, D), jnp.bfloat16),
        pltpu.VMEM_SHARED((N_AGG, D), jnp.bfloat16),
        pltpu.SemaphoreType.DMA,
        pltpu.SemaphoreType.DMA,
    ]

    @functools.partial(
        pl.kernel, mesh=_MESH,
        out_type=[jax.ShapeDtypeStruct((2, N_AGG, D), jnp.bfloat16)],
        compiler_params=pltpu.CompilerParams(use_tc_tiling_on_sc=False),
        scratch_types=scratch)
    def run(dsts_hbm, zrows_hbm, ones_hbm, cnt_hbm, dst_v, ones_v, acc_sh,
            sem0, sem1):
        cid = lax.axis_index("c")
        sid = lax.axis_index("s")
        wid = cid * 16 + sid

        pltpu.sync_copy(zrows_hbm, acc_sh.at[pl.ds(sid * SLAB_A, SLAB_A)])
        plsc.subcore_barrier()
        pltpu.sync_copy(ones_hbm, ones_v)

        def group(g, carry):
            pltpu.sync_copy(dsts_hbm.at[wid, pl.ds(g * GROUP, GROUP)], dst_v)
            _cnt_scatter(acc_sh, dst_v, ones_v, 0, sem0)
            _cnt_scatter(acc_sh, dst_v, ones_v, 1, sem1)

            def pair(p, c):
                _cnt_wait(acc_sh, dst_v, ones_v, sem0)
                _cnt_scatter(acc_sh, dst_v, ones_v, 2 * p + 2, sem0)
                _cnt_wait(acc_sh, dst_v, ones_v, sem1)
                _cnt_scatter(acc_sh, dst_v, ones_v, 2 * p + 3, sem1)
                return c

            lax.fori_loop(0, GROUP // 2 - 1, pair, carry)
            _cnt_wait(acc_sh, dst_v, ones_v, sem0)
            _cnt_wait(acc_sh, dst_v, ones_v, sem1)
            return carry

        lax.fori_loop(0, CHUNKS_PER_TILE // GROUP, group, 0)
        plsc.subcore_barrier()
        pltpu.sync_copy(acc_sh.at[pl.ds(sid * SLAB_A, SLAB_A)],
                        cnt_hbm.at[cid, pl.ds(sid * SLAB_A, SLAB_A)])

    return run(dsts, zrows_bf, ones)[0]


def _dense_body(relu, bf_out, agg_ref, cnt_ref, x_ref, wl_ref, wr_ref,
                b_ref, *o_refs):
    a = (agg_ref[0].astype(jnp.float32)
         + agg_ref[1].astype(jnp.float32))
    c = (cnt_ref[0, :, 0].astype(jnp.float32)
         + cnt_ref[1, :, 0].astype(jnp.float32))
    mean = a / jnp.maximum(c, 1.0)[:, None]
    h = lax.dot_general(mean, wl_ref[...], (((1,), (1,)), ((), ())),
                        preferred_element_type=jnp.float32)
    h = h + b_ref[...] + lax.dot_general(
        x_ref[...], wr_ref[...], (((1,), (1,)), ((), ())),
        preferred_element_type=jnp.float32)
    if relu:
        h = jnp.maximum(h, 0.0)
    o_refs[0][...] = h
    if bf_out:
        o_refs[1][...] = h.astype(jnp.bfloat16)


def _tc_dense(agg, cnt, x, Wl, Wr, b, relu, bf_out):
    R = 1000
    grid = (N_NODES // R,)
    out_shape = [jax.ShapeDtypeStruct((N_NODES, D), jnp.float32)]
    out_specs = [pl.BlockSpec((R, D), lambda i: (i, 0))]
    if bf_out:
        out_shape.append(jax.ShapeDtypeStruct((N_NODES, D), jnp.bfloat16))
        out_specs.append(pl.BlockSpec((R, D), lambda i: (i, 0)))
    return pl.pallas_call(
        functools.partial(_dense_body, relu, bf_out),
        grid=grid,
        in_specs=[
            pl.BlockSpec((2, R, D), lambda i: (0, i, 0)),
            pl.BlockSpec((2, R, D), lambda i: (0, i, 0)),
            pl.BlockSpec((R, D), lambda i: (i, 0)),
            pl.BlockSpec((D, D), lambda i: (0, 0)),
            pl.BlockSpec((D, D), lambda i: (0, 0)),
            pl.BlockSpec((1, D), lambda i: (0, 0)),
        ],
        out_specs=out_specs,
        out_shape=out_shape,
    )(agg, cnt, x, Wl, Wr, b.reshape(1, D))


def kernel(x, edge_index, W1l, b1, W1r, W2l, b2, W2r):
    src = edge_index[0].astype(jnp.int32)
    dst = edge_index[1].astype(jnp.int32)
    pad = E_PAD - N_EDGES
    srcs = jnp.concatenate([src, jnp.zeros((pad,), jnp.int32)]).reshape(
        NUM_TILES, CHUNKS_PER_TILE, CHUNK)
    dsts = jnp.concatenate(
        [dst, jnp.full((pad,), N_NODES, jnp.int32)]).reshape(
        NUM_TILES, CHUNKS_PER_TILE, CHUNK)
    ones = jnp.ones((CHUNK, D), jnp.bfloat16)
    zrows_bf = jnp.zeros((SLAB_A, D), jnp.bfloat16)
    rpad = ((0, N_AGG - N_NODES), (0, 0))

    cnt = _sc_count(dsts, zrows_bf, ones)
    aggx = _sc_aggregate(jnp.pad(x.astype(jnp.bfloat16), rpad),
                         srcs, dsts, zrows_bf)
    h1, h1bf = _tc_dense(aggx, cnt, x, W1l, W1r, b1, relu=True, bf_out=True)
    aggh = _sc_aggregate(jnp.pad(h1bf, rpad), srcs, dsts, zrows_bf)
    (out,) = _tc_dense(aggh, cnt, h1, W2l, W2r, b2, relu=False, bf_out=False)
    return out
